# baseline (speedup 1.0000x reference)
import jax
import jax.numpy as jnp
from jax.experimental import pallas as pl
from jax.experimental.pallas import tpu as pltpu


def _dot(a, b):
    return jax.lax.dot_general(a, b, (((1,), (0,)), ((), ())),
                               preferred_element_type=jnp.float32)


def _fused_kernel(x_ref, w1_ref, b1_ref, w2_ref, b2_ref, w3_ref, b3_ref,
                  wv1_ref, bv1_ref, wv2_ref, bv2_ref,
                  logits_ref, values_ref):
    x = x_ref[...]
    h1 = jnp.maximum(_dot(x, w1_ref[...]) + b1_ref[...], 0.0)
    h2 = jnp.maximum(_dot(h1, w2_ref[...]) + b2_ref[...], 0.0)
    logits_ref[...] = _dot(h2, w3_ref[...]) + b3_ref[...]
    v1 = jnp.maximum(_dot(x, wv1_ref[...]) + bv1_ref[...], 0.0)
    values_ref[...] = _dot(v1, wv2_ref[...]) + bv2_ref[...]


def kernel(hidden_states, W1, b1, W2, b2, W3, b3, Wv1, bv1, Wv2, bv2):
    B, S, H = hidden_states.shape
    N = B * S
    d1 = W1.shape[1]
    d2 = W2.shape[1]
    ne = W3.shape[1]

    flat = hidden_states.reshape(N, H)

    TILE = 2048
    grid = (N // TILE,)

    logits, values = pl.pallas_call(
        _fused_kernel,
        grid=grid,
        in_specs=[
            pl.BlockSpec((TILE, H), lambda i: (i, 0)),
            pl.BlockSpec((H, d1), lambda i: (0, 0)),
            pl.BlockSpec((1, d1), lambda i: (0, 0)),
            pl.BlockSpec((d1, d2), lambda i: (0, 0)),
            pl.BlockSpec((1, d2), lambda i: (0, 0)),
            pl.BlockSpec((d2, ne), lambda i: (0, 0)),
            pl.BlockSpec((1, ne), lambda i: (0, 0)),
            pl.BlockSpec((H, d1), lambda i: (0, 0)),
            pl.BlockSpec((1, d1), lambda i: (0, 0)),
            pl.BlockSpec((d1, 1), lambda i: (0, 0)),
            pl.BlockSpec((1, 1), lambda i: (0, 0)),
        ],
        out_specs=[
            pl.BlockSpec((TILE, ne), lambda i: (i, 0)),
            pl.BlockSpec((TILE, 1), lambda i: (i, 0)),
        ],
        out_shape=[
            jax.ShapeDtypeStruct((N, ne), jnp.float32),
            jax.ShapeDtypeStruct((N, 1), jnp.float32),
        ],
        compiler_params=pltpu.CompilerParams(
            dimension_semantics=("arbitrary",),
        ),
    )(flat, W1, b1.reshape(1, -1), W2, b2.reshape(1, -1),
      W3, b3.reshape(1, -1), Wv1, bv1.reshape(1, -1),
      Wv2, bv2.reshape(1, -1))

    return (logits.reshape(B, S, ne), values.reshape(B, S, 1))


# speedup vs baseline: 1.1898x; 1.1898x over previous
import jax
import jax.numpy as jnp
from jax.experimental import pallas as pl
from jax.experimental.pallas import tpu as pltpu


def _dot(a, b):
    return jax.lax.dot_general(a, b, (((1,), (0,)), ((), ())),
                               preferred_element_type=jnp.float32)


def _fused_kernel(x_ref, w1_ref, b1_ref, w2_ref, b2_ref, w3_ref, b3_ref,
                  wv1_ref, bv1_ref, wv2_ref, bv2_ref,
                  logits_ref, values_ref, wc_ref, *, d1):
    @pl.when(pl.program_id(0) == 0)
    def _init():
        wc_ref[:, :d1] = w1_ref[...].astype(jnp.bfloat16)
        wc_ref[:, d1:] = wv1_ref[...].astype(jnp.bfloat16)

    x = x_ref[...].astype(jnp.bfloat16)
    bc = jnp.concatenate([b1_ref[...], bv1_ref[...]], axis=1)
    hc = jnp.maximum(_dot(x, wc_ref[...]) + bc, 0.0)
    h1 = hc[:, :d1].astype(jnp.bfloat16)
    v1 = hc[:, d1:].astype(jnp.bfloat16)
    h2 = jnp.maximum(_dot(h1, w2_ref[...].astype(jnp.bfloat16)) + b2_ref[...],
                     0.0)
    logits_ref[...] = (_dot(h2.astype(jnp.bfloat16),
                            w3_ref[...].astype(jnp.bfloat16)) + b3_ref[...])
    values_ref[...] = (_dot(v1, wv2_ref[...].astype(jnp.bfloat16))
                       + bv2_ref[...])


def kernel(hidden_states, W1, b1, W2, b2, W3, b3, Wv1, bv1, Wv2, bv2):
    B, S, H = hidden_states.shape
    N = B * S
    d1 = W1.shape[1]
    d2 = W2.shape[1]
    ne = W3.shape[1]

    flat = hidden_states.reshape(N, H)

    TILE = 4096
    grid = (N // TILE,)

    import functools
    body = functools.partial(_fused_kernel, d1=d1)

    logits, values = pl.pallas_call(
        body,
        grid=grid,
        in_specs=[
            pl.BlockSpec((TILE, H), lambda i: (i, 0)),
            pl.BlockSpec((H, d1), lambda i: (0, 0)),
            pl.BlockSpec((1, d1), lambda i: (0, 0)),
            pl.BlockSpec((d1, d2), lambda i: (0, 0)),
            pl.BlockSpec((1, d2), lambda i: (0, 0)),
            pl.BlockSpec((d2, ne), lambda i: (0, 0)),
            pl.BlockSpec((1, ne), lambda i: (0, 0)),
            pl.BlockSpec((H, d1), lambda i: (0, 0)),
            pl.BlockSpec((1, d1), lambda i: (0, 0)),
            pl.BlockSpec((d1, 1), lambda i: (0, 0)),
            pl.BlockSpec((1, 1), lambda i: (0, 0)),
        ],
        out_specs=[
            pl.BlockSpec((TILE, ne), lambda i: (i, 0)),
            pl.BlockSpec((TILE, 1), lambda i: (i, 0)),
        ],
        out_shape=[
            jax.ShapeDtypeStruct((N, ne), jnp.float32),
            jax.ShapeDtypeStruct((N, 1), jnp.float32),
        ],
        scratch_shapes=[pltpu.VMEM((H, 2 * d1), jnp.bfloat16)],
        compiler_params=pltpu.CompilerParams(
            dimension_semantics=("arbitrary",),
        ),
    )(flat, W1, b1.reshape(1, -1), W2, b2.reshape(1, -1),
      W3, b3.reshape(1, -1), Wv1, bv1.reshape(1, -1),
      Wv2, bv2.reshape(1, -1))

    return (logits.reshape(B, S, ne), values.reshape(B, S, 1))
